# baseline (speedup 1.0000x reference)
import jax
import jax.numpy as jnp
from jax import lax
from jax.experimental import pallas as pl
from jax.experimental.pallas import tpu as pltpu
from jax.experimental.pallas import tpu_sc as plsc

_N = 10000
_D = 128
_H = 256
_C = 40
_E = 320000
_NP = 10240
_K = 128
_NCHT = _E // _K
_WCH = 80
_TCH = _NCHT - 31 * _WCH
_BCH = 40
_BN_EPS = 1e-5
_RPS = _NP // 16


def _sc_mesh():
    return plsc.VectorSubcoreMesh(
        core_axis_name="c", subcore_axis_name="s", num_cores=2, num_subcores=16
    )


def _deg_body(edges_hbm, tail_hbm, zeros_hbm, degs_hbm, acc, didx_all, vones):
    c = lax.axis_index("c")
    s = lax.axis_index("s")
    pltpu.sync_copy(zeros_hbm.at[pl.ds(s * _RPS, _RPS)], acc.at[pl.ds(s * _RPS, _RPS)])
    for i in range(_K // 16):
        vones[pl.ds(i * 16, 16)] = jnp.ones((16,), jnp.float32)
    plsc.subcore_barrier()
    w = c * 16 + s
    base = w * _WCH

    def scatter_ones(g0, g1):
        def chunk(g, c2):
            pltpu.sync_copy(vones, acc.at[didx_all.at[g]], add=True)
            return c2

        lax.fori_loop(g0, g1, chunk, 0)

    def batch(bi, carry):
        pltpu.sync_copy(edges_hbm.at[1, pl.ds(base + bi * _BCH, _BCH)], didx_all)
        scatter_ones(0, _BCH)
        return carry

    lax.fori_loop(0, jnp.where(w == 31, 0, _WCH // _BCH), batch, 0)

    @pl.when(w == 31)
    def _tail():
        pltpu.sync_copy(edges_hbm.at[1, pl.ds(base, 16)], didx_all.at[pl.ds(0, 16)])
        scatter_ones(0, 16)
        pltpu.sync_copy(tail_hbm.at[1], didx_all.at[pl.ds(0, 4)])
        scatter_ones(0, 4)

    plsc.subcore_barrier()
    pltpu.sync_copy(acc.at[pl.ds(s * _RPS, _RPS)], degs_hbm.at[c, pl.ds(s * _RPS, _RPS)])


def _run_deg(edges_r, tail_r):
    k = pl.kernel(
        _deg_body,
        out_type=jax.ShapeDtypeStruct((2, _NP), jnp.float32),
        mesh=_sc_mesh(),
        scratch_types=[
            pltpu.VMEM_SHARED((_NP,), jnp.float32),
            pltpu.VMEM((_BCH, _K), jnp.int32),
            pltpu.VMEM((_K,), jnp.float32),
        ],
    )
    return k(edges_r, tail_r, jnp.zeros((_NP,), jnp.float32))


def _agg_body(hb_hbm, edges_hbm, tail_hbm, out_hbm, acc, sidx_all, didx_all,
              row0, row1, row2, row3,
              sem0, sem1, sem2, sem3, sem4, sem5, sem6, sem7):
    c = lax.axis_index("c")
    s = lax.axis_index("s")
    pltpu.sync_copy(
        hb_hbm.at[pl.ds(s * _RPS, _RPS)], acc.at[pl.ds(s * _RPS, _RPS)]
    )
    plsc.subcore_barrier()
    rows = (row0, row1, row2, row3)
    gsems = (sem0, sem1, sem2, sem3)
    ssems = (sem4, sem5, sem6, sem7)
    w = c * 16 + s
    base = w * _WCH

    def src_at(j):
        return sidx_all.at[j >> 1, pl.ds((j & 1) * 64, 64)]

    def dst_at(j):
        return didx_all.at[j >> 1, pl.ds((j & 1) * 64, 64)]

    def pipe(n):
        for b in range(2):
            pltpu.async_copy(hb_hbm.at[src_at(b)], rows[b], gsems[b])

        def body(i, c2):
            for b in range(4):
                j = 4 * i + b
                b2 = (b + 2) % 4
                pltpu.make_async_copy(
                    hb_hbm.at[src_at(j)], rows[b], gsems[b]
                ).wait()
                pltpu.async_copy(rows[b], acc.at[dst_at(j)], ssems[b], add=True)

                @pl.when(j + 2 < n)
                def _issue():
                    @pl.when(j >= 2)
                    def _drain():
                        pltpu.make_async_copy(
                            rows[b2], acc.at[dst_at(j - 2)], ssems[b2]
                        ).wait()

                    pltpu.async_copy(
                        hb_hbm.at[src_at(j + 2)], rows[b2], gsems[b2]
                    )

            return c2

        lax.fori_loop(0, n // 4, body, 0)
        for b in range(4):
            pltpu.make_async_copy(rows[b], acc.at[dst_at(0)], ssems[b]).wait()

    def batch(bi, carry):
        cb = base + bi * _BCH
        pltpu.sync_copy(edges_hbm.at[0, pl.ds(cb, _BCH)], sidx_all)
        pltpu.sync_copy(edges_hbm.at[1, pl.ds(cb, _BCH)], didx_all)
        pipe(2 * _BCH)
        return carry

    lax.fori_loop(0, jnp.where(w == 31, 0, _WCH // _BCH), batch, 0)

    @pl.when(w == 31)
    def _tail():
        pltpu.sync_copy(edges_hbm.at[0, pl.ds(base, 16)], sidx_all.at[pl.ds(0, 16)])
        pltpu.sync_copy(edges_hbm.at[1, pl.ds(base, 16)], didx_all.at[pl.ds(0, 16)])
        pipe(32)
        pltpu.sync_copy(tail_hbm.at[0], sidx_all.at[pl.ds(0, 4)])
        pltpu.sync_copy(tail_hbm.at[1], didx_all.at[pl.ds(0, 4)])
        pipe(8)

    plsc.subcore_barrier()
    pltpu.sync_copy(
        acc.at[pl.ds(s * _RPS, _RPS)], out_hbm.at[c, pl.ds(s * _RPS, _RPS)]
    )


def _run_agg(table, edges_r, tail_r):
    k = pl.kernel(
        _agg_body,
        out_type=jax.ShapeDtypeStruct((2, _NP, 128), jnp.float32),
        mesh=_sc_mesh(),
        scratch_types=[
            pltpu.VMEM_SHARED((_NP, 128), jnp.float32),
            pltpu.VMEM((_BCH, _K), jnp.int32),
            pltpu.VMEM((_BCH, _K), jnp.int32),
            pltpu.VMEM((64, 128), jnp.float32),
            pltpu.VMEM((64, 128), jnp.float32),
            pltpu.VMEM((64, 128), jnp.float32),
            pltpu.VMEM((64, 128), jnp.float32),
            pltpu.SemaphoreType.DMA,
            pltpu.SemaphoreType.DMA,
            pltpu.SemaphoreType.DMA,
            pltpu.SemaphoreType.DMA,
            pltpu.SemaphoreType.DMA,
            pltpu.SemaphoreType.DMA,
            pltpu.SemaphoreType.DMA,
            pltpu.SemaphoreType.DMA,
        ],
    )
    return k(table, edges_r, tail_r)


_BN = 512


def _tca_body(x_ref, degs_ref, out_ref, dinv_ref):
    deg = 1.0 + degs_ref[0, :] + degs_ref[1, :]
    dinv = lax.rsqrt(deg)
    out_ref[...] = x_ref[...] * dinv[:, None]
    dinv_ref[...] = dinv[:, None]


def _run_tca(xp, degs):
    return pl.pallas_call(
        _tca_body,
        grid=(_NP // _BN,),
        in_specs=[
            pl.BlockSpec((_BN, _D), lambda i: (i, 0)),
            pl.BlockSpec((2, _BN), lambda i: (0, i)),
        ],
        out_specs=[
            pl.BlockSpec((_BN, _D), lambda i: (i, 0)),
            pl.BlockSpec((_BN, 1), lambda i: (i, 0)),
        ],
        out_shape=[
            jax.ShapeDtypeStruct((_NP, _D), jnp.float32),
            jax.ShapeDtypeStruct((_NP, 1), jnp.float32),
        ],
    )(xp, degs)


def _tcb_body(p_ref, xh_ref, dinv_ref, a_ref, bv_ref, w1_ref, w2_ref, out_ref):
    dinv = dinv_ref[...]
    u = (p_ref[0] + p_ref[1] - xh_ref[...]) * dinv
    h = jnp.dot(u, w1_ref[...], preferred_element_type=jnp.float32)
    h = jnp.maximum(h * a_ref[...] + bv_ref[...], 0.0)
    y = jnp.dot(h, w2_ref[...], preferred_element_type=jnp.float32)
    out_ref[...] = y * dinv


def _run_tcb(p, xhat, dinv, a_r, bv_r, W1, w2p):
    return pl.pallas_call(
        _tcb_body,
        grid=(_NP // _BN,),
        in_specs=[
            pl.BlockSpec((2, _BN, _D), lambda i: (0, i, 0)),
            pl.BlockSpec((_BN, _D), lambda i: (i, 0)),
            pl.BlockSpec((_BN, 1), lambda i: (i, 0)),
            pl.BlockSpec((1, _H), lambda i: (0, 0)),
            pl.BlockSpec((1, _H), lambda i: (0, 0)),
            pl.BlockSpec((_D, _H), lambda i: (0, 0)),
            pl.BlockSpec((_H, 128), lambda i: (0, 0)),
        ],
        out_specs=pl.BlockSpec((_BN, 128), lambda i: (i, 0)),
        out_shape=jax.ShapeDtypeStruct((_NP, 128), jnp.float32),
    )(p, xhat, dinv, a_r, bv_r, W1, w2p)


_BN3 = 2000


def _tcc_body(t_ref, yh_ref, dinv_ref, b2_ref, out_ref):
    dinv = dinv_ref[...]
    t = t_ref[...]
    o = (t[0] + t[1] - yh_ref[...])[:, : _C]
    o = o * dinv + b2_ref[...]
    m = jnp.max(o, axis=1, keepdims=True)
    l = o - m
    lse = jnp.log(jnp.sum(jnp.exp(l), axis=1, keepdims=True))
    out_ref[...] = l - lse


def _run_tcc(q, yhat, dinv, b2_r):
    return pl.pallas_call(
        _tcc_body,
        grid=(_N // _BN3,),
        in_specs=[
            pl.BlockSpec((2, _BN3, 128), lambda i: (0, i, 0)),
            pl.BlockSpec((_BN3, 128), lambda i: (i, 0)),
            pl.BlockSpec((_BN3, 1), lambda i: (i, 0)),
            pl.BlockSpec((1, _C), lambda i: (0, 0)),
        ],
        out_specs=pl.BlockSpec((_BN3, _C), lambda i: (i, 0)),
        out_shape=jax.ShapeDtypeStruct((_N, _C), jnp.float32),
    )(q, yhat, dinv, b2_r)


def kernel(x, edge_index, W1, b1, gamma1, beta1, rm1, rv1, W2, b2):
    edges_r = edge_index.reshape(2, _NCHT, _K)
    tail_r = edge_index[:, (_NCHT - 4) * _K:].reshape(2, 4, _K)
    xp = jnp.concatenate([x, jnp.zeros((_NP - _N, _D), jnp.float32)])

    degs = _run_deg(edges_r, tail_r)

    xhat, dinv = _run_tca(xp, degs)
    p = _run_agg(xhat, edges_r, tail_r)

    a = gamma1 * lax.rsqrt(rv1 + _BN_EPS)
    bv = (b1 - rm1) * a + beta1
    w2p = jnp.pad(W2, ((0, 0), (0, 128 - _C)))
    yhat = _run_tcb(p, xhat, dinv, a.reshape(1, _H), bv.reshape(1, _H), W1, w2p)
    q = _run_agg(yhat, edges_r, tail_r)

    return _run_tcc(q, yhat, dinv, b2.reshape(1, _C))


# speedup vs baseline: 1.2511x; 1.2511x over previous
import jax
import jax.numpy as jnp
from jax import lax
from jax.experimental import pallas as pl
from jax.experimental.pallas import tpu as pltpu
from jax.experimental.pallas import tpu_sc as plsc

_N = 10000
_D = 128
_H = 256
_C = 40
_E = 320000
_NP = 10240
_K = 128
_NCHT = _E // _K
_WCH = 80
_TCH = _NCHT - 31 * _WCH
_BCH = 40
_BN_EPS = 1e-5
_RPS = _NP // 16


def _sc_mesh():
    return plsc.VectorSubcoreMesh(
        core_axis_name="c", subcore_axis_name="s", num_cores=2, num_subcores=16
    )


def _deg_body(edges_hbm, tail_hbm, zeros_hbm, degs_hbm, acc, didx_all, vones,
              dsem):
    c = lax.axis_index("c")
    s = lax.axis_index("s")
    pltpu.sync_copy(zeros_hbm.at[pl.ds(s * _RPS, _RPS)], acc.at[pl.ds(s * _RPS, _RPS)])
    for i in range(_K // 16):
        vones[pl.ds(i * 16, 16)] = jnp.ones((16,), jnp.float32)
    plsc.subcore_barrier()
    w = c * 16 + s
    base = w * _WCH

    def scatter_ones(g0, g1):
        def fire(g, c2):
            pltpu.async_copy(vones, acc.at[didx_all.at[g]], dsem, add=True)
            return c2

        lax.fori_loop(g0, g1, fire, 0)

        def drain(g, c2):
            pltpu.make_async_copy(vones, acc.at[didx_all.at[g0]], dsem).wait()
            return c2

        lax.fori_loop(g0, g1, drain, 0)

    def batch(bi, carry):
        pltpu.sync_copy(edges_hbm.at[1, pl.ds(base + bi * _BCH, _BCH)], didx_all)
        scatter_ones(0, _BCH)
        return carry

    lax.fori_loop(0, jnp.where(w == 31, 0, _WCH // _BCH), batch, 0)

    @pl.when(w == 31)
    def _tail():
        pltpu.sync_copy(edges_hbm.at[1, pl.ds(base, 16)], didx_all.at[pl.ds(0, 16)])
        scatter_ones(0, 16)
        pltpu.sync_copy(tail_hbm.at[1], didx_all.at[pl.ds(0, 4)])
        scatter_ones(0, 4)

    plsc.subcore_barrier()
    pltpu.sync_copy(acc.at[pl.ds(s * _RPS, _RPS)], degs_hbm.at[c, pl.ds(s * _RPS, _RPS)])


def _run_deg(edges_r, tail_r):
    k = pl.kernel(
        _deg_body,
        out_type=jax.ShapeDtypeStruct((2, _NP), jnp.float32),
        mesh=_sc_mesh(),
        scratch_types=[
            pltpu.VMEM_SHARED((_NP,), jnp.float32),
            pltpu.VMEM((_BCH, _K), jnp.int32),
            pltpu.VMEM((_K,), jnp.float32),
            pltpu.SemaphoreType.DMA,
        ],
    )
    return k(edges_r, tail_r, jnp.zeros((_NP,), jnp.float32))


def _agg_body(hb_hbm, edges_hbm, tail_hbm, out_hbm, acc, sidx_all, didx_all,
              row0, row1, row2, row3, sem0, sem1, sem2, sem3):
    c = lax.axis_index("c")
    s = lax.axis_index("s")
    pltpu.sync_copy(
        hb_hbm.at[pl.ds(s * _RPS, _RPS)], acc.at[pl.ds(s * _RPS, _RPS)]
    )
    plsc.subcore_barrier()
    rows = (row0, row1, row2, row3)
    sems = (sem0, sem1, sem2, sem3)
    w = c * 16 + s
    base = w * _WCH

    def src_at(j):
        return sidx_all.at[j >> 1, pl.ds((j & 1) * 64, 64)]

    def dst_at(j):
        return didx_all.at[j >> 1, pl.ds((j & 1) * 64, 64)]

    def pipe(n):
        for b in range(4):
            pltpu.async_copy(hb_hbm.at[src_at(b)], rows[b], sems[b])

        def body(i, c2):
            for b in range(4):
                j = 4 * i + b
                pltpu.make_async_copy(
                    hb_hbm.at[src_at(j)], rows[b], sems[b]
                ).wait()
                pltpu.sync_copy(rows[b], acc.at[dst_at(j)], add=True)

                @pl.when(j + 4 < n)
                def _issue():
                    pltpu.async_copy(hb_hbm.at[src_at(j + 4)], rows[b], sems[b])

            return c2

        lax.fori_loop(0, n // 4, body, 0)

    def batch(bi, carry):
        cb = base + bi * _BCH
        pltpu.sync_copy(edges_hbm.at[0, pl.ds(cb, _BCH)], sidx_all)
        pltpu.sync_copy(edges_hbm.at[1, pl.ds(cb, _BCH)], didx_all)
        pipe(2 * _BCH)
        return carry

    lax.fori_loop(0, jnp.where(w == 31, 0, _WCH // _BCH), batch, 0)

    @pl.when(w == 31)
    def _tail():
        pltpu.sync_copy(edges_hbm.at[0, pl.ds(base, 16)], sidx_all.at[pl.ds(0, 16)])
        pltpu.sync_copy(edges_hbm.at[1, pl.ds(base, 16)], didx_all.at[pl.ds(0, 16)])
        pipe(32)
        pltpu.sync_copy(tail_hbm.at[0], sidx_all.at[pl.ds(0, 4)])
        pltpu.sync_copy(tail_hbm.at[1], didx_all.at[pl.ds(0, 4)])
        pipe(8)

    plsc.subcore_barrier()
    pltpu.sync_copy(
        acc.at[pl.ds(s * _RPS, _RPS)], out_hbm.at[c, pl.ds(s * _RPS, _RPS)]
    )


def _run_agg(table, edges_r, tail_r):
    k = pl.kernel(
        _agg_body,
        out_type=jax.ShapeDtypeStruct((2, _NP, 128), jnp.float32),
        mesh=_sc_mesh(),
        scratch_types=[
            pltpu.VMEM_SHARED((_NP, 128), jnp.float32),
            pltpu.VMEM((_BCH, _K), jnp.int32),
            pltpu.VMEM((_BCH, _K), jnp.int32),
            pltpu.VMEM((64, 128), jnp.float32),
            pltpu.VMEM((64, 128), jnp.float32),
            pltpu.VMEM((64, 128), jnp.float32),
            pltpu.VMEM((64, 128), jnp.float32),
            pltpu.SemaphoreType.DMA,
            pltpu.SemaphoreType.DMA,
            pltpu.SemaphoreType.DMA,
            pltpu.SemaphoreType.DMA,
        ],
    )
    return k(table, edges_r, tail_r)


_BN = 1024


def _tca_body(x_ref, degs_ref, out_ref, dinv_ref):
    deg = 1.0 + degs_ref[0, :] + degs_ref[1, :]
    dinv = lax.rsqrt(deg)
    out_ref[...] = x_ref[...] * dinv[:, None]
    dinv_ref[...] = dinv[:, None]


def _run_tca(xp, degs):
    return pl.pallas_call(
        _tca_body,
        grid=(_NP // _BN,),
        in_specs=[
            pl.BlockSpec((_BN, _D), lambda i: (i, 0)),
            pl.BlockSpec((2, _BN), lambda i: (0, i)),
        ],
        out_specs=[
            pl.BlockSpec((_BN, _D), lambda i: (i, 0)),
            pl.BlockSpec((_BN, 1), lambda i: (i, 0)),
        ],
        out_shape=[
            jax.ShapeDtypeStruct((_NP, _D), jnp.float32),
            jax.ShapeDtypeStruct((_NP, 1), jnp.float32),
        ],
    )(xp, degs)


def _tcb_body(p_ref, xh_ref, dinv_ref, a_ref, bv_ref, w1_ref, w2_ref, out_ref):
    dinv = dinv_ref[...]
    u = (p_ref[0] + p_ref[1] - xh_ref[...]) * dinv
    h = jnp.dot(u, w1_ref[...], preferred_element_type=jnp.float32)
    h = jnp.maximum(h * a_ref[...] + bv_ref[...], 0.0)
    y = jnp.dot(h, w2_ref[...], preferred_element_type=jnp.float32)
    out_ref[...] = y * dinv


def _run_tcb(p, xhat, dinv, a_r, bv_r, W1, w2p):
    return pl.pallas_call(
        _tcb_body,
        grid=(_NP // _BN,),
        in_specs=[
            pl.BlockSpec((2, _BN, _D), lambda i: (0, i, 0)),
            pl.BlockSpec((_BN, _D), lambda i: (i, 0)),
            pl.BlockSpec((_BN, 1), lambda i: (i, 0)),
            pl.BlockSpec((1, _H), lambda i: (0, 0)),
            pl.BlockSpec((1, _H), lambda i: (0, 0)),
            pl.BlockSpec((_D, _H), lambda i: (0, 0)),
            pl.BlockSpec((_H, 128), lambda i: (0, 0)),
        ],
        out_specs=pl.BlockSpec((_BN, 128), lambda i: (i, 0)),
        out_shape=jax.ShapeDtypeStruct((_NP, 128), jnp.float32),
    )(p, xhat, dinv, a_r, bv_r, W1, w2p)


_BN3 = 2000


def _tcc_body(t_ref, yh_ref, dinv_ref, b2_ref, out_ref):
    dinv = dinv_ref[...]
    t = t_ref[...]
    o = (t[0] + t[1] - yh_ref[...])[:, : _C]
    o = o * dinv + b2_ref[...]
    m = jnp.max(o, axis=1, keepdims=True)
    l = o - m
    lse = jnp.log(jnp.sum(jnp.exp(l), axis=1, keepdims=True))
    out_ref[...] = l - lse


def _run_tcc(q, yhat, dinv, b2_r):
    return pl.pallas_call(
        _tcc_body,
        grid=(_N // _BN3,),
        in_specs=[
            pl.BlockSpec((2, _BN3, 128), lambda i: (0, i, 0)),
            pl.BlockSpec((_BN3, 128), lambda i: (i, 0)),
            pl.BlockSpec((_BN3, 1), lambda i: (i, 0)),
            pl.BlockSpec((1, _C), lambda i: (0, 0)),
        ],
        out_specs=pl.BlockSpec((_BN3, _C), lambda i: (i, 0)),
        out_shape=jax.ShapeDtypeStruct((_N, _C), jnp.float32),
    )(q, yhat, dinv, b2_r)


def kernel(x, edge_index, W1, b1, gamma1, beta1, rm1, rv1, W2, b2):
    edges_r = edge_index.reshape(2, _NCHT, _K)
    tail_r = edge_index[:, (_NCHT - 4) * _K:].reshape(2, 4, _K)
    xp = jnp.concatenate([x, jnp.zeros((_NP - _N, _D), jnp.float32)])

    degs = _run_deg(edges_r, tail_r)

    xhat, dinv = _run_tca(xp, degs)
    p = _run_agg(xhat, edges_r, tail_r)

    a = gamma1 * lax.rsqrt(rv1 + _BN_EPS)
    bv = (b1 - rm1) * a + beta1
    w2p = jnp.pad(W2, ((0, 0), (0, 128 - _C)))
    yhat = _run_tcb(p, xhat, dinv, a.reshape(1, _H), bv.reshape(1, _H), W1, w2p)
    q = _run_agg(yhat, edges_r, tail_r)

    return _run_tcc(q, yhat, dinv, b2.reshape(1, _C))
